# baseline (speedup 1.0000x reference)
import functools

import jax
import jax.numpy as jnp
from jax import lax
from jax.experimental import pallas as pl

E = 8
TOP_K = 2
HIDDEN = 768
INTER = 3072
T = 2048
LANES = 128
IC_BLK = 1024
NEG = -1e30


def _router_body(x_ref, rwt_ref, coef_ref, aux_ref):
    x = x_ref[...].astype(jnp.bfloat16)
    rwt = rwt_ref[...].astype(jnp.bfloat16)
    logits = jnp.dot(x, rwt, preferred_element_type=jnp.float32)
    lane = lax.broadcasted_iota(jnp.int32, (T, LANES), 1)
    valid = lane < E
    logits = jnp.where(valid, logits, NEG)
    m = jnp.max(logits, axis=1, keepdims=True)
    p = jnp.exp(logits - m)
    probs = p / jnp.sum(p, axis=1, keepdims=True)
    m1 = jnp.max(probs, axis=1, keepdims=True)
    i1 = jnp.min(jnp.where(probs == m1, lane, LANES), axis=1, keepdims=True)
    sel1 = lane == i1
    probs_m = jnp.where(sel1, -1.0, probs)
    m2 = jnp.max(probs_m, axis=1, keepdims=True)
    i2 = jnp.min(jnp.where(probs_m == m2, lane, LANES), axis=1, keepdims=True)
    sel2 = lane == i2
    s = m1 + m2
    coef = jnp.where(sel1, m1 / s, 0.0) + jnp.where(sel2, m2 / s, 0.0)
    coef_ref[...] = coef
    counts = jnp.sum((sel1 | sel2).astype(jnp.float32), axis=0,
                     keepdims=True)
    psum = jnp.sum(probs, axis=0, keepdims=True)
    aux_ref[...] = (E / (T * T)) * jnp.sum(counts * psum, axis=1,
                                           keepdims=True)


def _moe_body(xb_ref, w1_ref, w2_ref, coef_ref, out_ref):
    e = pl.program_id(0)
    j = pl.program_id(1)
    inter = jnp.dot(xb_ref[...], w1_ref[0],
                    preferred_element_type=jnp.float32)
    sig = 1.0 / (1.0 + jnp.exp(-inter))
    act = (inter * sig).astype(jnp.bfloat16)
    partial = jnp.dot(act, w2_ref[0],
                      preferred_element_type=jnp.float32)
    lane = lax.broadcasted_iota(jnp.int32, (T, LANES), 1)
    col = jnp.sum(jnp.where(lane == e, coef_ref[...], 0.0),
                  axis=1, keepdims=True)

    @pl.when(jnp.logical_and(e == 0, j == 0))
    def _():
        out_ref[...] = jnp.zeros_like(out_ref)

    out_ref[...] += col * partial


@jax.jit
def kernel(x, router_w, expert_w1, expert_w2):
    orig_shape = x.shape
    xf = x.reshape(-1, HIDDEN)
    rwt = jnp.zeros((HIDDEN, LANES), jnp.float32).at[:, :E].set(router_w.T)

    coef, aux = pl.pallas_call(
        _router_body,
        out_shape=[
            jax.ShapeDtypeStruct((T, LANES), jnp.float32),
            jax.ShapeDtypeStruct((1, 1), jnp.float32),
        ],
    )(xf, rwt)

    xb = xf.astype(jnp.bfloat16)
    w1b = expert_w1.astype(jnp.bfloat16)
    w2b = expert_w2.astype(jnp.bfloat16)
    nj = INTER // IC_BLK

    out = pl.pallas_call(
        _moe_body,
        grid=(E, nj),
        in_specs=[
            pl.BlockSpec((T, HIDDEN), lambda e, j: (0, 0)),
            pl.BlockSpec((1, HIDDEN, IC_BLK), lambda e, j: (e, 0, j)),
            pl.BlockSpec((1, IC_BLK, HIDDEN), lambda e, j: (e, j, 0)),
            pl.BlockSpec((T, LANES), lambda e, j: (0, 0)),
        ],
        out_specs=pl.BlockSpec((T, HIDDEN), lambda e, j: (0, 0)),
        out_shape=jax.ShapeDtypeStruct((T, HIDDEN), jnp.float32),
    )(xb, w1b, w2b, coef)

    return out.reshape(orig_shape), aux.reshape(())


# speedup vs baseline: 1.2216x; 1.2216x over previous
import functools

import jax
import jax.numpy as jnp
from jax import lax
from jax.experimental import pallas as pl
from jax.experimental.pallas import tpu as pltpu
from jax.experimental.pallas import tpu_sc as plsc

E = 8
TOP_K = 2
HIDDEN = 768
INTER = 3072
T = 2048
P = T * TOP_K
LANES = 128
B = 256
NB = P // B + E
NSLOT = NB * B
CHUNK = 512
NCH = P // CHUNK
NJ = 2
NEG = -1e30
NW = 32
PPW = P // NW
TPW = T // NW


def _router_dispatch_body(x_ref, rwt_ref, slot_ref, c1_ref, c2_ref,
                          be_ref, nb_ref, aux_ref):
    x = x_ref[...].astype(jnp.bfloat16)
    rwt = rwt_ref[...].astype(jnp.bfloat16)
    logits = jnp.dot(x, rwt, preferred_element_type=jnp.float32)
    lane = lax.broadcasted_iota(jnp.int32, (T, LANES), 1)
    logits = jnp.where(lane < E, logits, NEG)
    m = jnp.max(logits, axis=1, keepdims=True)
    p = jnp.exp(logits - m)
    probs = p / jnp.sum(p, axis=1, keepdims=True)
    m1 = jnp.max(probs, axis=1, keepdims=True)
    i1 = jnp.min(jnp.where(probs == m1, lane, LANES), axis=1, keepdims=True)
    sel1 = lane == i1
    probs_m = jnp.where(sel1, -1.0, probs)
    m2 = jnp.max(probs_m, axis=1, keepdims=True)
    i2 = jnp.min(jnp.where(probs_m == m2, lane, LANES), axis=1, keepdims=True)
    sel2 = lane == i2
    s = m1 + m2
    c1_ref[...] = jnp.broadcast_to(m1 / s, (T, 16))
    c2_ref[...] = jnp.broadcast_to(m2 / s, (T, 16))
    oh1 = sel1.astype(jnp.float32)
    oh2 = sel2.astype(jnp.float32)
    counts = jnp.sum(oh1, axis=0, keepdims=True) + \
        jnp.sum(oh2, axis=0, keepdims=True)
    psum = jnp.sum(probs, axis=0, keepdims=True)
    aux_ref[...] = (E / (T * T)) * jnp.sum(counts * psum, axis=1,
                                           keepdims=True)
    hi = lax.Precision.HIGHEST
    padded = jnp.ceil(counts / B) * B
    lt = (lax.broadcasted_iota(jnp.int32, (LANES, LANES), 0) <
          lax.broadcasted_iota(jnp.int32, (LANES, LANES), 1)
          ).astype(jnp.float32)
    astart = jnp.dot(padded, lt, preferred_element_type=jnp.float32,
                     precision=hi)
    ltc = (lax.broadcasted_iota(jnp.int32, (CHUNK, CHUNK), 0) >
           lax.broadcasted_iota(jnp.int32, (CHUNK, CHUNK), 1)
           ).astype(jnp.float32)

    base = jnp.zeros((1, LANES), jnp.float32)
    for c in range(NCH):
        k = c // (T // CHUNK)
        trow = c * CHUNK - k * T
        ohsrc = oh1 if k == 0 else oh2
        oh = ohsrc[trow:trow + CHUNK, :]
        cum = jnp.dot(ltc, oh, preferred_element_type=jnp.float32,
                      precision=hi)
        pos = cum + base + astart
        sl = jnp.sum(oh * pos, axis=1, keepdims=True)
        slot_ref[c * CHUNK:(c + 1) * CHUNK, :] = sl.astype(jnp.int32)
        base = base + jnp.sum(oh, axis=0, keepdims=True)
    bstart = astart / B
    irow = lax.broadcasted_iota(jnp.int32, (LANES, LANES), 0
                                ).astype(jnp.float32)
    bmask = jnp.logical_and(bstart <= irow, lane[:LANES, :] < E)
    be_ref[...] = (jnp.sum(bmask.astype(jnp.float32), axis=1,
                           keepdims=True) - 1.0).astype(jnp.int32)
    nb_ref[...] = (jnp.sum(padded, axis=1, keepdims=True) / B).astype(jnp.int32)


def _mlp_body(meta_ref, xs_ref, w1_ref, w2_ref, ys_ref):
    i = pl.program_id(0)
    j = pl.program_id(1)

    @pl.when(i < meta_ref[0])
    def _():
        xb = xs_ref[...].astype(jnp.bfloat16)
        inter = jnp.dot(xb, w1_ref[0].astype(jnp.bfloat16),
                        preferred_element_type=jnp.float32)
        act = (inter * (1.0 / (1.0 + jnp.exp(-inter)))).astype(jnp.bfloat16)
        partial = jnp.dot(act, w2_ref[0].astype(jnp.bfloat16),
                          preferred_element_type=jnp.float32)

        @pl.when(j == 0)
        def _():
            ys_ref[...] = partial

        @pl.when(j != 0)
        def _():
            ys_ref[...] += partial


def _sc_dispatch_body(tok_hbm, slot_hbm, xf_hbm, xs_hbm,
                      tok_v, slot_v, rows_v, sem):
    wid = lax.axis_index("s") * 2 + lax.axis_index("c")
    base = wid * PPW
    pltpu.sync_copy(tok_hbm.at[pl.ds(base, PPW)], tok_v)
    pltpu.sync_copy(slot_hbm.at[pl.ds(base, PPW)], slot_v)
    pltpu.async_copy(xf_hbm.at[tok_v], rows_v, sem).wait()
    pltpu.async_copy(rows_v, xs_hbm.at[slot_v], sem).wait()


def _sc_combine_body(slot_hbm, c1_hbm, c2_hbm, ys_hbm, out_hbm,
                     s0_v, s1_v, g0_v, g1_v, c1_v, c2_v, sem):
    wid = lax.axis_index("s") * 2 + lax.axis_index("c")
    base = wid * TPW
    pltpu.sync_copy(slot_hbm.at[pl.ds(base, TPW)], s0_v)
    pltpu.sync_copy(slot_hbm.at[pl.ds(T + base, TPW)], s1_v)
    pltpu.sync_copy(c1_hbm.at[pl.ds(base, TPW)], c1_v)
    pltpu.sync_copy(c2_hbm.at[pl.ds(base, TPW)], c2_v)
    pltpu.async_copy(ys_hbm.at[s0_v], g0_v, sem).wait()
    pltpu.async_copy(ys_hbm.at[s1_v], g1_v, sem).wait()

    def tok_step(t, _):
        a = c1_v[t, :]
        b = c2_v[t, :]

        def ch_step(ch, _):
            off = ch * 16
            g0_v[t, pl.ds(off, 16)] = (a * g0_v[t, pl.ds(off, 16)] +
                                       b * g1_v[t, pl.ds(off, 16)])
            return 0

        return lax.fori_loop(0, HIDDEN // 16, ch_step, 0)

    lax.fori_loop(0, TPW, tok_step, 0)
    pltpu.sync_copy(g0_v, out_hbm.at[pl.ds(base, TPW)])


@functools.cache
def _sc_kernels():
    mesh = plsc.VectorSubcoreMesh(core_axis_name="c", subcore_axis_name="s")
    dispatch = pl.kernel(
        _sc_dispatch_body, mesh=mesh,
        out_type=jax.ShapeDtypeStruct((NSLOT, HIDDEN), jnp.float32),
        scratch_types=[
            pltpu.VMEM((PPW,), jnp.int32),
            pltpu.VMEM((PPW,), jnp.int32),
            pltpu.VMEM((PPW, HIDDEN), jnp.float32),
            pltpu.SemaphoreType.DMA,
        ],
    )
    combine = pl.kernel(
        _sc_combine_body, mesh=mesh,
        out_type=jax.ShapeDtypeStruct((T, HIDDEN), jnp.float32),
        scratch_types=[
            pltpu.VMEM((TPW,), jnp.int32),
            pltpu.VMEM((TPW,), jnp.int32),
            pltpu.VMEM((TPW, HIDDEN), jnp.float32),
            pltpu.VMEM((TPW, HIDDEN), jnp.float32),
            pltpu.VMEM((TPW, 16), jnp.float32),
            pltpu.VMEM((TPW, 16), jnp.float32),
            pltpu.SemaphoreType.DMA,
        ],
    )
    return dispatch, combine


@jax.jit
def kernel(x, router_w, expert_w1, expert_w2):
    orig_shape = x.shape
    xf = x.reshape(-1, HIDDEN)
    rwt = jnp.zeros((HIDDEN, LANES), jnp.float32).at[:, :E].set(router_w.T)

    slot2d, c1, c2, be2d, nb2d, aux = pl.pallas_call(
        _router_dispatch_body,
        out_shape=[
            jax.ShapeDtypeStruct((P, 1), jnp.int32),
            jax.ShapeDtypeStruct((T, 16), jnp.float32),
            jax.ShapeDtypeStruct((T, 16), jnp.float32),
            jax.ShapeDtypeStruct((LANES, 1), jnp.int32),
            jax.ShapeDtypeStruct((1, 1), jnp.int32),
            jax.ShapeDtypeStruct((1, 1), jnp.float32),
        ],
    )(xf, rwt)

    slot = slot2d.reshape(P)
    meta = jnp.concatenate([nb2d.reshape(1), be2d.reshape(LANES)[:NB]])

    tokid = jnp.tile(jnp.arange(T, dtype=jnp.int32), TOP_K)
    sc_dispatch, sc_combine = _sc_kernels()
    xs = sc_dispatch(tokid, slot, xf)

    ys = pl.pallas_call(
        _mlp_body,
        grid_spec=pltpu.PrefetchScalarGridSpec(
            num_scalar_prefetch=1,
            grid=(NB, NJ),
            in_specs=[
                pl.BlockSpec((B, HIDDEN), lambda i, j, m: (i, 0)),
                pl.BlockSpec((1, HIDDEN, INTER // NJ),
                             lambda i, j, m: (m[i + 1], 0, j)),
                pl.BlockSpec((1, INTER // NJ, HIDDEN),
                             lambda i, j, m: (m[i + 1], j, 0)),
            ],
            out_specs=pl.BlockSpec((B, HIDDEN), lambda i, j, m: (i, 0)),
        ),
        out_shape=jax.ShapeDtypeStruct((NSLOT, HIDDEN), jnp.float32),
    )(meta, xs, expert_w1, expert_w2)

    out = sc_combine(slot, c1, c2, ys)
    return out.reshape(orig_shape), aux.reshape(())


# speedup vs baseline: 1.6704x; 1.3673x over previous
import functools

import jax
import jax.numpy as jnp
from jax import lax
from jax.experimental import pallas as pl
from jax.experimental.pallas import tpu as pltpu
from jax.experimental.pallas import tpu_sc as plsc

E = 8
TOP_K = 2
HIDDEN = 768
INTER = 3072
T = 2048
P = T * TOP_K
LANES = 128
B = 256
NB = P // B + E
NSLOT = NB * B
CHUNK = 512
NCH = P // CHUNK
NJ = 1
NEG = -1e30
NW = 32
PPW = P // NW
TPW = T // NW


def _router_dispatch_body(x_ref, rwt_ref, slot_ref, c1_ref, c2_ref,
                          be_ref, nb_ref, aux_ref):
    x = x_ref[...].astype(jnp.bfloat16)
    rwt = rwt_ref[...].astype(jnp.bfloat16)
    logits = jnp.dot(x, rwt, preferred_element_type=jnp.float32)
    lane = lax.broadcasted_iota(jnp.int32, (T, LANES), 1)
    logits = jnp.where(lane < E, logits, NEG)
    m = jnp.max(logits, axis=1, keepdims=True)
    p = jnp.exp(logits - m)
    probs = p / jnp.sum(p, axis=1, keepdims=True)
    m1 = jnp.max(probs, axis=1, keepdims=True)
    i1 = jnp.min(jnp.where(probs == m1, lane, LANES), axis=1, keepdims=True)
    sel1 = lane == i1
    probs_m = jnp.where(sel1, -1.0, probs)
    m2 = jnp.max(probs_m, axis=1, keepdims=True)
    i2 = jnp.min(jnp.where(probs_m == m2, lane, LANES), axis=1, keepdims=True)
    sel2 = lane == i2
    s = m1 + m2
    c1_ref[...] = jnp.broadcast_to(m1 / s, (T, 16))
    c2_ref[...] = jnp.broadcast_to(m2 / s, (T, 16))
    oh1 = sel1.astype(jnp.float32)
    oh2 = sel2.astype(jnp.float32)
    counts = jnp.sum(oh1, axis=0, keepdims=True) + \
        jnp.sum(oh2, axis=0, keepdims=True)
    psum = jnp.sum(probs, axis=0, keepdims=True)
    aux_ref[...] = (E / (T * T)) * jnp.sum(counts * psum, axis=1,
                                           keepdims=True)
    hi = lax.Precision.HIGHEST
    padded = jnp.ceil(counts / B) * B
    lt = (lax.broadcasted_iota(jnp.int32, (LANES, LANES), 0) <
          lax.broadcasted_iota(jnp.int32, (LANES, LANES), 1)
          ).astype(jnp.float32)
    astart = jnp.dot(padded, lt, preferred_element_type=jnp.float32,
                     precision=hi)
    ltc = (lax.broadcasted_iota(jnp.int32, (CHUNK, CHUNK), 0) >
           lax.broadcasted_iota(jnp.int32, (CHUNK, CHUNK), 1)
           ).astype(jnp.float32)

    base = jnp.zeros((1, LANES), jnp.float32)
    for c in range(NCH):
        k = c // (T // CHUNK)
        trow = c * CHUNK - k * T
        ohsrc = oh1 if k == 0 else oh2
        oh = ohsrc[trow:trow + CHUNK, :]
        cum = jnp.dot(ltc, oh, preferred_element_type=jnp.float32,
                      precision=hi)
        pos = cum + base + astart
        sl = jnp.sum(oh * pos, axis=1, keepdims=True)
        slot_ref[c * CHUNK:(c + 1) * CHUNK, :] = sl.astype(jnp.int32)
        base = base + jnp.sum(oh, axis=0, keepdims=True)
    bstart = astart / B
    irow = lax.broadcasted_iota(jnp.int32, (LANES, LANES), 0
                                ).astype(jnp.float32)
    bmask = jnp.logical_and(bstart <= irow, lane[:LANES, :] < E)
    be_ref[...] = (jnp.sum(bmask.astype(jnp.float32), axis=1,
                           keepdims=True) - 1.0).astype(jnp.int32)
    nb_ref[...] = (jnp.sum(padded, axis=1, keepdims=True) / B).astype(jnp.int32)


def _mlp_body(meta_ref, xs_ref, w1_ref, w2_ref, ys_ref):
    i = pl.program_id(0)

    @pl.when(i < meta_ref[0])
    def _():
        xb = xs_ref[...].astype(jnp.bfloat16)
        inter = jnp.dot(xb, w1_ref[0].astype(jnp.bfloat16),
                        preferred_element_type=jnp.float32)
        act = (inter * (1.0 / (1.0 + jnp.exp(-inter)))).astype(jnp.bfloat16)
        ys_ref[...] = jnp.dot(act, w2_ref[0].astype(jnp.bfloat16),
                              preferred_element_type=jnp.float32)


def _sc_dispatch_body(tok_hbm, slot_hbm, xf_hbm, xs_hbm,
                      tok_v, slot_v, rows_v, sem):
    wid = lax.axis_index("s") * 2 + lax.axis_index("c")
    base = wid * PPW
    pltpu.sync_copy(tok_hbm.at[pl.ds(base, PPW)], tok_v)
    pltpu.sync_copy(slot_hbm.at[pl.ds(base, PPW)], slot_v)
    pltpu.async_copy(xf_hbm.at[tok_v], rows_v, sem).wait()
    pltpu.async_copy(rows_v, xs_hbm.at[slot_v], sem).wait()


def _sc_combine_body(slot_hbm, c1_hbm, c2_hbm, ys_hbm, out_hbm,
                     s0_v, s1_v, g0_v, g1_v, c1_v, c2_v, sem):
    wid = lax.axis_index("s") * 2 + lax.axis_index("c")
    base = wid * TPW
    pltpu.sync_copy(slot_hbm.at[pl.ds(base, TPW)], s0_v)
    pltpu.sync_copy(slot_hbm.at[pl.ds(T + base, TPW)], s1_v)
    pltpu.sync_copy(c1_hbm.at[pl.ds(base, TPW)], c1_v)
    pltpu.sync_copy(c2_hbm.at[pl.ds(base, TPW)], c2_v)
    cp0 = pltpu.async_copy(ys_hbm.at[s0_v], g0_v, sem)
    cp1 = pltpu.async_copy(ys_hbm.at[s1_v], g1_v, sem)
    cp0.wait()
    cp1.wait()

    def tok_step(t, _):
        a = c1_v[t, :]
        b = c2_v[t, :]
        for ch in range(HIDDEN // 16):
            off = ch * 16
            g0_v[t, pl.ds(off, 16)] = (a * g0_v[t, pl.ds(off, 16)] +
                                       b * g1_v[t, pl.ds(off, 16)])
        return 0

    lax.fori_loop(0, TPW, tok_step, 0)
    pltpu.sync_copy(g0_v, out_hbm.at[pl.ds(base, TPW)])


@functools.cache
def _sc_kernels():
    mesh = plsc.VectorSubcoreMesh(core_axis_name="c", subcore_axis_name="s")
    dispatch = pl.kernel(
        _sc_dispatch_body, mesh=mesh,
        out_type=jax.ShapeDtypeStruct((NSLOT, HIDDEN), jnp.float32),
        scratch_types=[
            pltpu.VMEM((PPW,), jnp.int32),
            pltpu.VMEM((PPW,), jnp.int32),
            pltpu.VMEM((PPW, HIDDEN), jnp.float32),
            pltpu.SemaphoreType.DMA,
        ],
    )
    combine = pl.kernel(
        _sc_combine_body, mesh=mesh,
        out_type=jax.ShapeDtypeStruct((T, HIDDEN), jnp.float32),
        scratch_types=[
            pltpu.VMEM((TPW,), jnp.int32),
            pltpu.VMEM((TPW,), jnp.int32),
            pltpu.VMEM((TPW, HIDDEN), jnp.float32),
            pltpu.VMEM((TPW, HIDDEN), jnp.float32),
            pltpu.VMEM((TPW, 16), jnp.float32),
            pltpu.VMEM((TPW, 16), jnp.float32),
            pltpu.SemaphoreType.DMA,
        ],
    )
    return dispatch, combine


@jax.jit
def kernel(x, router_w, expert_w1, expert_w2):
    orig_shape = x.shape
    xf = x.reshape(-1, HIDDEN)
    rwt = jnp.zeros((HIDDEN, LANES), jnp.float32).at[:, :E].set(router_w.T)

    slot2d, c1, c2, be2d, nb2d, aux = pl.pallas_call(
        _router_dispatch_body,
        out_shape=[
            jax.ShapeDtypeStruct((P, 1), jnp.int32),
            jax.ShapeDtypeStruct((T, 16), jnp.float32),
            jax.ShapeDtypeStruct((T, 16), jnp.float32),
            jax.ShapeDtypeStruct((LANES, 1), jnp.int32),
            jax.ShapeDtypeStruct((1, 1), jnp.int32),
            jax.ShapeDtypeStruct((1, 1), jnp.float32),
        ],
    )(xf, rwt)

    slot = slot2d.reshape(P)
    meta = jnp.concatenate([nb2d.reshape(1), be2d.reshape(LANES)[:NB]])

    tokid = jnp.tile(jnp.arange(T, dtype=jnp.int32), TOP_K)
    sc_dispatch, sc_combine = _sc_kernels()
    xs = sc_dispatch(tokid, slot, xf)

    ys = pl.pallas_call(
        _mlp_body,
        grid_spec=pltpu.PrefetchScalarGridSpec(
            num_scalar_prefetch=1,
            grid=(NB,),
            in_specs=[
                pl.BlockSpec((B, HIDDEN), lambda i, m: (i, 0)),
                pl.BlockSpec((1, HIDDEN, INTER),
                             lambda i, m: (m[i + 1], 0, 0)),
                pl.BlockSpec((1, INTER, HIDDEN),
                             lambda i, m: (m[i + 1], 0, 0)),
            ],
            out_specs=pl.BlockSpec((B, HIDDEN), lambda i, m: (i, 0)),
        ),
        out_shape=jax.ShapeDtypeStruct((NSLOT, HIDDEN), jnp.float32),
    )(meta, xs, expert_w1, expert_w2)

    out = sc_combine(slot, c1, c2, ys)
    return out.reshape(orig_shape), aux.reshape(())


# speedup vs baseline: 1.7333x; 1.0377x over previous
import functools

import jax
import jax.numpy as jnp
from jax import lax
from jax.experimental import pallas as pl
from jax.experimental.pallas import tpu as pltpu
from jax.experimental.pallas import tpu_sc as plsc

E = 8
TOP_K = 2
HIDDEN = 768
INTER = 3072
T = 2048
P = T * TOP_K
LANES = 128
B = 256
NB = P // B + E
NSLOT = NB * B
CHUNK = 512
NCH = P // CHUNK
NJ = 1
NEG = -1e30
NW = 32
PPW = P // NW
TPW = T // NW


def _router_dispatch_body(x_ref, rwt_ref, slot_ref, c1_ref, c2_ref,
                          be_ref, nb_ref, aux_ref):
    x = x_ref[...].astype(jnp.bfloat16)
    rwt = rwt_ref[...].astype(jnp.bfloat16)
    logits = jnp.dot(x, rwt, preferred_element_type=jnp.float32)
    lane = lax.broadcasted_iota(jnp.int32, (T, LANES), 1)
    logits = jnp.where(lane < E, logits, NEG)
    m = jnp.max(logits, axis=1, keepdims=True)
    p = jnp.exp(logits - m)
    probs = p / jnp.sum(p, axis=1, keepdims=True)
    m1 = jnp.max(probs, axis=1, keepdims=True)
    i1 = jnp.min(jnp.where(probs == m1, lane, LANES), axis=1, keepdims=True)
    sel1 = lane == i1
    probs_m = jnp.where(sel1, -1.0, probs)
    m2 = jnp.max(probs_m, axis=1, keepdims=True)
    i2 = jnp.min(jnp.where(probs_m == m2, lane, LANES), axis=1, keepdims=True)
    sel2 = lane == i2
    s = m1 + m2
    c1_ref[...] = jnp.broadcast_to(m1 / s, (T, 16))
    c2_ref[...] = jnp.broadcast_to(m2 / s, (T, 16))
    oh1 = sel1.astype(jnp.float32)
    oh2 = sel2.astype(jnp.float32)
    counts = jnp.sum(oh1, axis=0, keepdims=True) + \
        jnp.sum(oh2, axis=0, keepdims=True)
    psum = jnp.sum(probs, axis=0, keepdims=True)
    aux_ref[...] = (E / (T * T)) * jnp.sum(counts * psum, axis=1,
                                           keepdims=True)
    padded = jnp.ceil(counts / B) * B
    lt = (lax.broadcasted_iota(jnp.int32, (LANES, LANES), 0) <
          lax.broadcasted_iota(jnp.int32, (LANES, LANES), 1)
          ).astype(jnp.bfloat16)
    astart = jnp.dot(padded.astype(jnp.bfloat16), lt,
                     preferred_element_type=jnp.float32)
    ltc = (lax.broadcasted_iota(jnp.int32, (CHUNK, CHUNK), 0) >
           lax.broadcasted_iota(jnp.int32, (CHUNK, CHUNK), 1)
           ).astype(jnp.bfloat16)

    base = jnp.zeros((1, LANES), jnp.float32)
    for c in range(NCH):
        k = c // (T // CHUNK)
        trow = c * CHUNK - k * T
        ohsrc = oh1 if k == 0 else oh2
        oh = ohsrc[trow:trow + CHUNK, :]
        cum = jnp.dot(ltc, oh.astype(jnp.bfloat16),
                      preferred_element_type=jnp.float32)
        pos = cum + base + astart
        sl = jnp.sum(oh * pos, axis=1, keepdims=True)
        slot_ref[c * CHUNK:(c + 1) * CHUNK, :] = sl.astype(jnp.int32)
        base = base + jnp.sum(oh, axis=0, keepdims=True)
    bstart = astart / B
    irow = lax.broadcasted_iota(jnp.int32, (LANES, LANES), 0
                                ).astype(jnp.float32)
    bmask = jnp.logical_and(bstart <= irow, lane[:LANES, :] < E)
    be_ref[...] = (jnp.sum(bmask.astype(jnp.float32), axis=1,
                           keepdims=True) - 1.0).astype(jnp.int32)
    nb_ref[...] = (jnp.sum(padded, axis=1, keepdims=True) / B).astype(jnp.int32)


def _mlp_body(meta_ref, xs_ref, w1_ref, w2_ref, ys_ref):
    i = pl.program_id(0)

    @pl.when(i < meta_ref[0])
    def _():
        xb = xs_ref[...].astype(jnp.bfloat16)
        inter = jnp.dot(xb, w1_ref[0].astype(jnp.bfloat16),
                        preferred_element_type=jnp.float32)
        act = (inter * (1.0 / (1.0 + jnp.exp(-inter)))).astype(jnp.bfloat16)
        ys_ref[...] = jnp.dot(act, w2_ref[0].astype(jnp.bfloat16),
                              preferred_element_type=jnp.float32)


def _sc_dispatch_body(slot_hbm, xf_hbm, xs_hbm,
                      tok_a, tok_b, slot_a, slot_b, rows_a, rows_b,
                      sem_a, sem_b, sem_s):
    wid = lax.axis_index("s") * 2 + lax.axis_index("c")
    base = wid * PPW
    half = PPW // 2
    tbase = (wid % (T // PPW)) * PPW
    for j in range(half // 16):
        tok_a[pl.ds(j * 16, 16)] = tbase + j * 16 + lax.iota(jnp.int32, 16)
        tok_b[pl.ds(j * 16, 16)] = (tbase + half + j * 16 +
                                    lax.iota(jnp.int32, 16))
    pltpu.sync_copy(slot_hbm.at[pl.ds(base, half)], slot_a)
    pltpu.sync_copy(slot_hbm.at[pl.ds(base + half, half)], slot_b)
    ga = pltpu.async_copy(xf_hbm.at[tok_a], rows_a, sem_a)
    gb = pltpu.async_copy(xf_hbm.at[tok_b], rows_b, sem_b)
    ga.wait()
    sa = pltpu.async_copy(rows_a, xs_hbm.at[slot_a], sem_s)
    gb.wait()
    sb = pltpu.async_copy(rows_b, xs_hbm.at[slot_b], sem_s)
    sa.wait()
    sb.wait()


def _sc_combine_body(slot_hbm, c1_hbm, c2_hbm, ys_hbm, out_hbm,
                     s0a, s0b, s1a, s1b, g0_v, g1_v, c1_v, c2_v,
                     sem_a, sem_b, sem_o):
    wid = lax.axis_index("s") * 2 + lax.axis_index("c")
    base = wid * TPW
    half = TPW // 2
    pltpu.sync_copy(slot_hbm.at[pl.ds(base, half)], s0a)
    pltpu.sync_copy(slot_hbm.at[pl.ds(base + half, half)], s0b)
    pltpu.sync_copy(slot_hbm.at[pl.ds(T + base, half)], s1a)
    pltpu.sync_copy(slot_hbm.at[pl.ds(T + base + half, half)], s1b)
    pltpu.sync_copy(c1_hbm.at[pl.ds(base, TPW)], c1_v)
    pltpu.sync_copy(c2_hbm.at[pl.ds(base, TPW)], c2_v)
    g0a = pltpu.async_copy(ys_hbm.at[s0a], g0_v.at[pl.ds(0, half)], sem_a)
    g1a = pltpu.async_copy(ys_hbm.at[s1a], g1_v.at[pl.ds(0, half)], sem_a)
    g0b = pltpu.async_copy(ys_hbm.at[s0b], g0_v.at[pl.ds(half, half)], sem_b)
    g1b = pltpu.async_copy(ys_hbm.at[s1b], g1_v.at[pl.ds(half, half)], sem_b)

    def tok_step(t, _):
        a = c1_v[t, :]
        b = c2_v[t, :]
        for ch in range(HIDDEN // 16):
            off = ch * 16
            g0_v[t, pl.ds(off, 16)] = (a * g0_v[t, pl.ds(off, 16)] +
                                       b * g1_v[t, pl.ds(off, 16)])
        return 0

    g0a.wait()
    g1a.wait()
    lax.fori_loop(0, half, tok_step, 0)
    oa = pltpu.async_copy(g0_v.at[pl.ds(0, half)],
                          out_hbm.at[pl.ds(base, half)], sem_o)
    g0b.wait()
    g1b.wait()
    lax.fori_loop(half, TPW, tok_step, 0)
    ob = pltpu.async_copy(g0_v.at[pl.ds(half, half)],
                          out_hbm.at[pl.ds(base + half, half)], sem_o)
    oa.wait()
    ob.wait()


@functools.cache
def _sc_kernels():
    mesh = plsc.VectorSubcoreMesh(core_axis_name="c", subcore_axis_name="s")
    dispatch = pl.kernel(
        _sc_dispatch_body, mesh=mesh,
        out_type=jax.ShapeDtypeStruct((NSLOT, HIDDEN), jnp.float32),
        scratch_types=[
            pltpu.VMEM((PPW // 2,), jnp.int32),
            pltpu.VMEM((PPW // 2,), jnp.int32),
            pltpu.VMEM((PPW // 2,), jnp.int32),
            pltpu.VMEM((PPW // 2,), jnp.int32),
            pltpu.VMEM((PPW // 2, HIDDEN), jnp.float32),
            pltpu.VMEM((PPW // 2, HIDDEN), jnp.float32),
            pltpu.SemaphoreType.DMA,
            pltpu.SemaphoreType.DMA,
            pltpu.SemaphoreType.DMA,
        ],
    )
    combine = pl.kernel(
        _sc_combine_body, mesh=mesh,
        out_type=jax.ShapeDtypeStruct((T, HIDDEN), jnp.float32),
        scratch_types=[
            pltpu.VMEM((TPW // 2,), jnp.int32),
            pltpu.VMEM((TPW // 2,), jnp.int32),
            pltpu.VMEM((TPW // 2,), jnp.int32),
            pltpu.VMEM((TPW // 2,), jnp.int32),
            pltpu.VMEM((TPW, HIDDEN), jnp.float32),
            pltpu.VMEM((TPW, HIDDEN), jnp.float32),
            pltpu.VMEM((TPW, 16), jnp.float32),
            pltpu.VMEM((TPW, 16), jnp.float32),
            pltpu.SemaphoreType.DMA,
            pltpu.SemaphoreType.DMA,
            pltpu.SemaphoreType.DMA,
        ],
    )
    return dispatch, combine


@jax.jit
def kernel(x, router_w, expert_w1, expert_w2):
    orig_shape = x.shape
    xf = x.reshape(-1, HIDDEN)
    rwt = jnp.zeros((HIDDEN, LANES), jnp.float32).at[:, :E].set(router_w.T)

    slot2d, c1, c2, be2d, nb2d, aux = pl.pallas_call(
        _router_dispatch_body,
        out_shape=[
            jax.ShapeDtypeStruct((P, 1), jnp.int32),
            jax.ShapeDtypeStruct((T, 16), jnp.float32),
            jax.ShapeDtypeStruct((T, 16), jnp.float32),
            jax.ShapeDtypeStruct((LANES, 1), jnp.int32),
            jax.ShapeDtypeStruct((1, 1), jnp.int32),
            jax.ShapeDtypeStruct((1, 1), jnp.float32),
        ],
    )(xf, rwt)

    slot = slot2d.reshape(P)
    meta = jnp.concatenate([nb2d.reshape(1), be2d.reshape(LANES)[:NB]])

    sc_dispatch, sc_combine = _sc_kernels()
    xs = sc_dispatch(slot, xf)

    ys = pl.pallas_call(
        _mlp_body,
        grid_spec=pltpu.PrefetchScalarGridSpec(
            num_scalar_prefetch=1,
            grid=(NB,),
            in_specs=[
                pl.BlockSpec((B, HIDDEN), lambda i, m: (i, 0)),
                pl.BlockSpec((1, HIDDEN, INTER),
                             lambda i, m: (m[i + 1], 0, 0)),
                pl.BlockSpec((1, INTER, HIDDEN),
                             lambda i, m: (m[i + 1], 0, 0)),
            ],
            out_specs=pl.BlockSpec((B, HIDDEN), lambda i, m: (i, 0)),
        ),
        out_shape=jax.ShapeDtypeStruct((NSLOT, HIDDEN), jnp.float32),
    )(meta, xs, expert_w1, expert_w2)

    out = sc_combine(slot, c1, c2, ys)
    return out.reshape(orig_shape), aux.reshape(())
